# Initial kernel scaffold; baseline (speedup 1.0000x reference)
import jax
import jax.numpy as jnp
from jax.experimental import pallas as pl


def kernel(feat_trainable, feat_criterion, dec_masks):
    raise NotImplementedError("write your pallas kernel here")


# speedup vs baseline: 10.4541x; 10.4541x over previous
import jax
import jax.numpy as jnp
from jax.experimental import pallas as pl
from jax.experimental.pallas import tpu as pltpu

_TEMP = 0.1
_BASE_TEMP = 0.07
_TOPK = 16
_G = 8
_GN = 2048
_C = 128
_RB = 256
_NRB = _GN // _RB
_SCALE = -(_TEMP / _BASE_TEMP) / (_G * _GN * 2.0)


def _fused_body(fc_ref, ft_ref, dm_ref, out_ref, fcn, ftn, lab):
    gi = pl.program_id(0)
    rb = pl.program_id(1)

    @pl.when(rb == 0)
    def _prep():
        fc = fc_ref[0]
        nc = jnp.sqrt(jnp.sum(fc * fc, axis=-1, keepdims=True))
        fcn[...] = fc / jnp.maximum(nc, 1e-12)
        ft = ft_ref[0]
        nt = jnp.sqrt(jnp.sum(ft * ft, axis=-1, keepdims=True))
        ftn[...] = ft / jnp.maximum(nt, 1e-12)
        dm = dm_ref[0]
        mx = jnp.max(dm, axis=1, keepdims=True)
        sidx = jax.lax.broadcasted_iota(jnp.int32, dm.shape, 1)
        cand = jnp.where(dm == mx, sidx, dm.shape[1])
        lab[...] = jnp.min(cand, axis=1).reshape(1, _GN).astype(jnp.float32)

    r0 = rb * _RB
    fcn_all = fcn[...]
    ftn_all = ftn[...]
    fcb = fcn[pl.ds(r0, _RB), :]
    sim = jax.lax.dot_general(
        fcb, fcn_all, (((1,), (1,)), ((), ())),
        preferred_element_type=jnp.float32)

    neg = jnp.float32(-jnp.inf)
    t = jnp.full((_RB, 1), jnp.inf, jnp.float32)
    thr = jnp.zeros((_RB, 1), jnp.float32)
    done = jnp.zeros((_RB, 1), jnp.bool_)
    for _ in range(_TOPK):
        m = jnp.max(jnp.where(sim < t, sim, neg), axis=1, keepdims=True)
        c = jnp.sum(jnp.where(sim >= m, 1.0, 0.0), axis=1, keepdims=True)
        reach = c >= _TOPK
        hit = jnp.logical_and(jnp.logical_not(done), reach)
        thr = jnp.where(hit, m, thr)
        done = jnp.logical_or(done, reach)
        t = jnp.where(done, t, m)

    ftb = ftn[pl.ds(r0, _RB), :]
    logits = jax.lax.dot_general(
        ftb, ftn_all, (((1,), (1,)), ((), ())),
        preferred_element_type=jnp.float32) / _TEMP

    col = jax.lax.broadcasted_iota(jnp.int32, (_RB, _GN), 1)
    row = jax.lax.broadcasted_iota(jnp.int32, (_RB, _GN), 0) + r0
    offd = jnp.where(col != row, 1.0, 0.0)

    el = jnp.exp(logits) * offd
    denom = jnp.sum(el, axis=1, keepdims=True)
    log_prob = logits - jnp.log(denom)

    pos = jnp.where(sim >= thr, offd, 0.0)
    pos_sum = jnp.sum(pos * log_prob, axis=1, keepdims=True)
    pos_cnt = jnp.sum(pos, axis=1, keepdims=True)

    labr = lab[:, pl.ds(r0, _RB)].reshape(_RB, 1)
    labc = lab[...]
    semi = jnp.where(labr == labc, offd, 0.0)
    semi_sum = jnp.sum(semi * log_prob, axis=1, keepdims=True)
    semi_cnt = jnp.sum(semi, axis=1, keepdims=True)

    row_loss = (pos_sum / (pos_cnt + 1e-8)
                + semi_sum / (semi_cnt + 1e-8))
    contrib = (jnp.sum(row_loss) * _SCALE).reshape(1, 1)

    @pl.when(jnp.logical_and(gi == 0, rb == 0))
    def _init():
        out_ref[...] = jnp.zeros_like(out_ref)

    out_ref[...] += contrib


def kernel(feat_trainable, feat_criterion, dec_masks):
    ft = feat_trainable.reshape(_G, _GN, _C)
    fc = feat_criterion.reshape(_G, _GN, _C)
    dm = dec_masks.reshape(_G, 2, 16, 1024)

    out = pl.pallas_call(
        _fused_body,
        grid=(_G, _NRB),
        in_specs=[
            pl.BlockSpec((1, _GN, _C), lambda gi, rb: (gi, 0, 0)),
            pl.BlockSpec((1, _GN, _C), lambda gi, rb: (gi, 0, 0)),
            pl.BlockSpec((1, 2, 16, 1024), lambda gi, rb: (gi, 0, 0, 0)),
        ],
        out_specs=pl.BlockSpec((1, 1), lambda gi, rb: (0, 0)),
        out_shape=jax.ShapeDtypeStruct((1, 1), jnp.float32),
        scratch_shapes=[
            pltpu.VMEM((_GN, _C), jnp.float32),
            pltpu.VMEM((_GN, _C), jnp.float32),
            pltpu.VMEM((1, _GN), jnp.float32),
        ],
    )(fc, ft, dm)
    return out[0, 0]
